# Initial kernel scaffold; baseline (speedup 1.0000x reference)
import jax
import jax.numpy as jnp
from jax.experimental import pallas as pl


def kernel(node_logits, edge_logits, edge_index, W_proj, b_proj, W_e1, b_e1, W_e2, b_e2, W_ap, b_ap, W_v, b_v, W_o, b_o, W_c1, b_c1, W_c2, b_c2, W_nh, b_nh, W_eh, b_eh):
    raise NotImplementedError("write your pallas kernel here")


# speedup vs baseline: 2.5394x; 2.5394x over previous
import functools

import jax
import jax.numpy as jnp
from jax import lax
from jax.experimental import pallas as pl
from jax.experimental.pallas import tpu as pltpu
from jax.experimental.pallas import tpu_sc as plsc

N = 50000
E = 800000
H = 32
NC = 4
EC = 4

SC_CORES = 2
SC_SUBCORES = 16
SC_WORKERS = SC_CORES * SC_SUBCORES
LANES = 16

K_EDGES = 200
SUB = 100
NSUB = K_EDGES // SUB
NCHUNKS = E // K_EDGES
T_STEPS = -(-NCHUNKS // SC_WORKERS)
ROWS_PER_TILE = 3128
ROWS_LAST = N - (SC_SUBCORES - 1) * ROWS_PER_TILE

_P = jax.lax.Precision.HIGHEST


def _dot(x, w):
    return jnp.dot(x, w, precision=_P, preferred_element_type=jnp.float32)


def _fold_body(W_ap, b_ap, W_v, b_v, W_o, b_o, Ws, Wd, bc):
    wvo = _dot(W_v[...], W_o[...])
    Ws[...] = _dot(W_ap[pl.ds(0, H), :], wvo)
    Wd[...] = _dot(W_ap[pl.ds(H, H), :], wvo)
    bc[...] = _dot(_dot(b_ap[...], W_v[...]) + b_v[...], W_o[...]) + b_o[...]


def _fold_weights(W_ap, b_ap, W_v, b_v, W_o, b_o):
    full = lambda s: pl.BlockSpec(s, lambda: (0,) * len(s))
    return pl.pallas_call(
        _fold_body,
        out_shape=(jax.ShapeDtypeStruct((H, H), jnp.float32),
                   jax.ShapeDtypeStruct((H, H), jnp.float32),
                   jax.ShapeDtypeStruct((1, H), jnp.float32)),
        in_specs=[full((2 * H, H)), full((1, H)), full((H, H)), full((1, H)),
                  full((H, H)), full((1, H))],
        out_specs=(full((H, H)), full((H, H)), full((1, H))),
    )(W_ap, b_ap.reshape(1, H), W_v, b_v.reshape(1, H), W_o, b_o.reshape(1, H))


def _prep_node_body(nl, W_proj, b_proj, Ws, Wd, bc, S, D):
    nf = _dot(nl[...], W_proj[...]) + b_proj[...]
    S[:, pl.ds(0, H)] = _dot(nf, Ws[...])
    S[:, pl.ds(H, H)] = nf
    D[...] = _dot(nf, Wd[...]) + bc[...]


def _prep_node(node_logits, W_proj, b_proj, Ws, Wd, bc):
    RN = 2000
    g = N // RN
    full = lambda s: pl.BlockSpec(s, lambda i: (0,) * len(s))
    return pl.pallas_call(
        _prep_node_body,
        grid=(g,),
        out_shape=(jax.ShapeDtypeStruct((N, 2 * H), jnp.float32),
                   jax.ShapeDtypeStruct((N, H), jnp.float32)),
        in_specs=[pl.BlockSpec((RN, NC), lambda i: (i, 0)),
                  full((NC, H)), full((1, H)), full((H, H)), full((H, H)),
                  full((1, H))],
        out_specs=(pl.BlockSpec((RN, 2 * H), lambda i: (i, 0)),
                   pl.BlockSpec((RN, H), lambda i: (i, 0))),
    )(node_logits, W_proj, b_proj.reshape(1, H), Ws, Wd, bc)


def _prep_edge_body(el, W_e1, b_e1, W_e2, b_e2, ef):
    t = jnp.maximum(_dot(el[...], W_e1[...]) + b_e1[...], 0.0)
    ef[...] = _dot(t, W_e2[...]) + b_e2[...]


def _prep_edge(edge_logits, W_e1, b_e1, W_e2, b_e2):
    RE = 8000
    g = E // RE
    full = lambda s: pl.BlockSpec(s, lambda i: (0,) * len(s))
    return pl.pallas_call(
        _prep_edge_body,
        grid=(g,),
        out_shape=jax.ShapeDtypeStruct((E, H), jnp.float32),
        in_specs=[pl.BlockSpec((RE, EC), lambda i: (i, 0)),
                  full((EC, H)), full((1, H)), full((H, H)), full((1, H))],
        out_specs=pl.BlockSpec((RE, H), lambda i: (i, 0)),
    )(edge_logits, W_e1, b_e1.reshape(1, H), W_e2, b_e2.reshape(1, H))


def _node_update_body(S_in, agg0, agg1, W_c1, b_c1, W_c2, b_c2, Ws, Wd, bc,
                      S, D):
    nf = S_in[:, pl.ds(H, H)]
    h = nf + agg0[...] + agg1[...]
    t = jnp.maximum(_dot(h, W_c1[...]) + b_c1[...], 0.0)
    nf2 = jnp.maximum(_dot(t, W_c2[...]) + b_c2[...], 0.0)
    S[:, pl.ds(0, H)] = _dot(nf2, Ws[...])
    S[:, pl.ds(H, H)] = nf2
    D[...] = _dot(nf2, Wd[...]) + bc[...]


def _node_update(S_in, agg, W_c1, b_c1, W_c2, b_c2, Ws, Wd, bc):
    RN = 2000
    g = N // RN
    full = lambda s: pl.BlockSpec(s, lambda i: (0,) * len(s))
    return pl.pallas_call(
        _node_update_body,
        grid=(g,),
        out_shape=(jax.ShapeDtypeStruct((N, 2 * H), jnp.float32),
                   jax.ShapeDtypeStruct((N, H), jnp.float32)),
        in_specs=[pl.BlockSpec((RN, 2 * H), lambda i: (i, 0)),
                  pl.BlockSpec((RN, H), lambda i: (i, 0)),
                  pl.BlockSpec((RN, H), lambda i: (i, 0)),
                  full((H, H)), full((1, H)), full((H, H)), full((1, H)),
                  full((H, H)), full((H, H)), full((1, H))],
        out_specs=(pl.BlockSpec((RN, 2 * H), lambda i: (i, 0)),
                   pl.BlockSpec((RN, H), lambda i: (i, 0))),
    )(S_in, agg[0], agg[1], W_c1, b_c1.reshape(1, H), W_c2,
      b_c2.reshape(1, H), Ws, Wd, bc)


def _node_final_body(S_in, agg0, agg1, W_c1, b_c1, W_c2, b_c2, W_nh, b_nh,
                     out):
    nf = S_in[:, pl.ds(H, H)]
    h = nf + agg0[...] + agg1[...]
    t = jnp.maximum(_dot(h, W_c1[...]) + b_c1[...], 0.0)
    nf2 = jnp.maximum(_dot(t, W_c2[...]) + b_c2[...], 0.0)
    out[...] = _dot(nf2, W_nh[...]) + b_nh[...]


def _node_final(S_in, agg, W_c1, b_c1, W_c2, b_c2, W_nh, b_nh):
    RN = 2000
    g = N // RN
    full = lambda s: pl.BlockSpec(s, lambda i: (0,) * len(s))
    return pl.pallas_call(
        _node_final_body,
        grid=(g,),
        out_shape=jax.ShapeDtypeStruct((N, NC), jnp.float32),
        in_specs=[pl.BlockSpec((RN, 2 * H), lambda i: (i, 0)),
                  pl.BlockSpec((RN, H), lambda i: (i, 0)),
                  pl.BlockSpec((RN, H), lambda i: (i, 0)),
                  full((H, H)), full((1, H)), full((H, H)), full((1, H)),
                  full((H, NC)), full((1, NC))],
        out_specs=pl.BlockSpec((RN, NC), lambda i: (i, 0)),
    )(S_in, agg[0], agg[1], W_c1, b_c1.reshape(1, H), W_c2,
      b_c2.reshape(1, H), W_nh, b_nh.reshape(1, NC))


def _edge_head_body(ef, W_eh, b_eh, out):
    out[...] = _dot(ef[...], W_eh[...]) + b_eh[...]


def _edge_head(ef, W_eh, b_eh):
    RE = 8000
    g = E // RE
    full = lambda s: pl.BlockSpec(s, lambda i: (0,) * len(s))
    return pl.pallas_call(
        _edge_head_body,
        grid=(g,),
        out_shape=jax.ShapeDtypeStruct((E, EC), jnp.float32),
        in_specs=[pl.BlockSpec((RE, H), lambda i: (i, 0)),
                  full((H, EC)), full((1, EC))],
        out_specs=pl.BlockSpec((RE, EC), lambda i: (i, 0)),
    )(ef, W_eh, b_eh.reshape(1, EC))


def _sc_edge_body(S_hbm, D_hbm, ef_hbm, src_hbm, dst_hbm,
                  efo_hbm, agg_hbm,
                  sidx, didx, efb, sb, db, aggsh, sem):
    c = lax.axis_index("c")
    s = lax.axis_index("s")
    w = s * SC_CORES + c

    zero = jnp.zeros((LANES,), jnp.float32)

    def _zrow(e, _):
        db[e, pl.ds(0, LANES)] = zero
        db[e, pl.ds(LANES, LANES)] = zero
        return _

    lax.fori_loop(0, K_EDGES, _zrow, None)
    base = s * ROWS_PER_TILE

    def _zero_rows(nrows):
        off = 0
        while off < nrows:
            sz = min(K_EDGES, nrows - off)
            pltpu.sync_copy(db.at[pl.ds(0, sz)],
                            aggsh.at[pl.ds(base + off, sz)])
            off += sz

    @pl.when(s < SC_SUBCORES - 1)
    def _():
        _zero_rows(ROWS_PER_TILE)

    @pl.when(s == SC_SUBCORES - 1)
    def _():
        _zero_rows(ROWS_LAST)

    plsc.subcore_barrier()

    def _chunk(t, _):
        chunk = w + t * SC_WORKERS

        @pl.when(chunk < NCHUNKS)
        def _():
            ebase = chunk * K_EDGES
            d_si = pltpu.async_copy(src_hbm.at[chunk], sidx, sem)
            d_di = pltpu.async_copy(dst_hbm.at[chunk], didx, sem)
            d_ef = pltpu.async_copy(ef_hbm.at[pl.ds(ebase, K_EDGES)], efb, sem)
            d_si.wait()
            d_di.wait()
            gathers = []
            for j in range(NSUB):
                gathers.append(pltpu.async_copy(
                    S_hbm.at[sidx.at[j]], sb.at[pl.ds(j * SUB, SUB)], sem))
                gathers.append(pltpu.async_copy(
                    D_hbm.at[didx.at[j]], db.at[pl.ds(j * SUB, SUB)], sem))
            d_ef.wait()
            for g in gathers:
                g.wait()

            def _edge(e, _):
                a0 = sb[e, pl.ds(0, LANES)]
                a1 = sb[e, pl.ds(LANES, LANES)]
                f0 = efb[e, pl.ds(0, LANES)]
                f1 = efb[e, pl.ds(LANES, LANES)]
                d0 = db[e, pl.ds(0, LANES)]
                d1 = db[e, pl.ds(LANES, LANES)]
                g0 = jnp.maximum(f0 + a0 + d0, 0.0)
                g1 = jnp.maximum(f1 + a1 + d1, 0.0)
                efb[e, pl.ds(0, LANES)] = g0
                efb[e, pl.ds(LANES, LANES)] = g1
                c0 = sb[e, pl.ds(2 * LANES, LANES)]
                c1 = sb[e, pl.ds(3 * LANES, LANES)]
                db[e, pl.ds(0, LANES)] = jnp.maximum(c0 + g0, 0.0)
                db[e, pl.ds(LANES, LANES)] = jnp.maximum(c1 + g1, 0.0)
                return _

            lax.fori_loop(0, K_EDGES, _edge, None)

            for j in range(NSUB):
                pltpu.sync_copy(db.at[pl.ds(j * SUB, SUB)],
                                aggsh.at[didx.at[j]], add=True)
            pltpu.sync_copy(efb, efo_hbm.at[pl.ds(ebase, K_EDGES)])
        return _

    lax.fori_loop(0, T_STEPS, _chunk, None)
    plsc.subcore_barrier()

    @pl.when(s < SC_SUBCORES - 1)
    def _():
        pltpu.sync_copy(aggsh.at[pl.ds(base, ROWS_PER_TILE)],
                        agg_hbm.at[c, pl.ds(base, ROWS_PER_TILE)])

    @pl.when(s == SC_SUBCORES - 1)
    def _():
        pltpu.sync_copy(aggsh.at[pl.ds(base, ROWS_LAST)],
                        agg_hbm.at[c, pl.ds(base, ROWS_LAST)])


def _sc_edge_pass(S_tab, D_tab, ef, src2, dst2):
    mesh = plsc.VectorSubcoreMesh(core_axis_name="c", subcore_axis_name="s")
    return pl.kernel(
        _sc_edge_body,
        out_type=(jax.ShapeDtypeStruct((E, H), jnp.float32),
                  jax.ShapeDtypeStruct((SC_CORES, N, H), jnp.float32)),
        mesh=mesh,
        compiler_params=pltpu.CompilerParams(use_tc_tiling_on_sc=False),
        scratch_types=[
            pltpu.VMEM((NSUB, SUB), jnp.int32),
            pltpu.VMEM((NSUB, SUB), jnp.int32),
            pltpu.VMEM((K_EDGES, H), jnp.float32),
            pltpu.VMEM((K_EDGES, 2 * H), jnp.float32),
            pltpu.VMEM((K_EDGES, H), jnp.float32),
            pltpu.VMEM_SHARED((N, H), jnp.float32),
            pltpu.SemaphoreType.DMA,
        ],
    )(S_tab, D_tab, ef, src2, dst2)


def kernel(node_logits, edge_logits, edge_index, W_proj, b_proj, W_e1, b_e1,
           W_e2, b_e2, W_ap, b_ap, W_v, b_v, W_o, b_o, W_c1, b_c1, W_c2,
           b_c2, W_nh, b_nh, W_eh, b_eh):
    src2 = edge_index[0].reshape(NCHUNKS, NSUB, SUB)
    dst2 = edge_index[1].reshape(NCHUNKS, NSUB, SUB)

    Ws, Wd, bc = _fold_weights(W_ap, b_ap, W_v, b_v, W_o, b_o)
    S, D = _prep_node(node_logits, W_proj, b_proj, Ws, Wd, bc)
    ef = _prep_edge(edge_logits, W_e1, b_e1, W_e2, b_e2)

    ef, agg = _sc_edge_pass(S, D, ef, src2, dst2)
    S, D = _node_update(S, agg, W_c1, b_c1, W_c2, b_c2, Ws, Wd, bc)

    ef, agg = _sc_edge_pass(S, D, ef, src2, dst2)
    node_out = _node_final(S, agg, W_c1, b_c1, W_c2, b_c2, W_nh, b_nh)
    edge_out = _edge_head(ef, W_eh, b_eh)
    return (node_out, edge_out)


# speedup vs baseline: 3.6805x; 1.4493x over previous
import jax
import jax.numpy as jnp
from jax import lax
from jax.experimental import pallas as pl
from jax.experimental.pallas import tpu as pltpu
from jax.experimental.pallas import tpu_sc as plsc

N = 50000
E = 800000
H = 32
NC = 4
EC = 4

SC_CORES = 2
SC_SUBCORES = 16
SC_WORKERS = SC_CORES * SC_SUBCORES
LANES = 16

K_EDGES = 200
SUB = 100
NSUB = K_EDGES // SUB
NCHUNKS = E // K_EDGES
T_STEPS = -(-NCHUNKS // SC_WORKERS)
ROWS_PER_TILE = 3128
ROWS_LAST = N - (SC_SUBCORES - 1) * ROWS_PER_TILE

_P = jax.lax.Precision.HIGHEST


def _dot(x, w):
    return jnp.dot(x, w, precision=_P, preferred_element_type=jnp.float32)


def _kron(w, p):
    return jnp.kron(jnp.eye(p, dtype=jnp.float32), w)


def _tileb(b, p):
    return jnp.tile(b.reshape(-1), p).reshape(1, -1)


def _fold_body(W_ap, b_ap, W_v, b_v, W_o, b_o, Ws, Wd, bc):
    wvo = _dot(W_v[...], W_o[...])
    Ws[...] = _dot(W_ap[pl.ds(0, H), :], wvo)
    Wd[...] = _dot(W_ap[pl.ds(H, H), :], wvo)
    bc[...] = _dot(_dot(b_ap[...], W_v[...]) + b_v[...], W_o[...]) + b_o[...]


def _fold_weights(W_ap, b_ap, W_v, b_v, W_o, b_o):
    full = lambda s: pl.BlockSpec(s, lambda: (0,) * len(s))
    return pl.pallas_call(
        _fold_body,
        out_shape=(jax.ShapeDtypeStruct((H, H), jnp.float32),
                   jax.ShapeDtypeStruct((H, H), jnp.float32),
                   jax.ShapeDtypeStruct((1, H), jnp.float32)),
        in_specs=[full((2 * H, H)), full((1, H)), full((H, H)), full((1, H)),
                  full((H, H)), full((1, H))],
        out_specs=(full((H, H)), full((H, H)), full((1, H))),
    )(W_ap, b_ap.reshape(1, H), W_v, b_v.reshape(1, H), W_o, b_o.reshape(1, H))


NB = 4
NRB = N // 4 // NB


def _prep_node_body(nl4, WpK, bpK, WsK, WdK, bcK, nf4, a4, d4):
    nf = _dot(nl4[0], WpK[...]) + bpK[...]
    nf4[0] = nf
    a4[0] = _dot(nf, WsK[...])
    d4[0] = _dot(nf, WdK[...]) + bcK[...]


def _prep_node(nl4, WpK, bpK, WsK, WdK, bcK):
    full = lambda s: pl.BlockSpec(s, lambda i: (0,) * len(s))
    blk = lambda m: pl.BlockSpec((1, NRB, m), lambda i: (i, 0, 0))
    o = jax.ShapeDtypeStruct((NB, NRB, 128), jnp.float32)
    return pl.pallas_call(
        _prep_node_body,
        grid=(NB,),
        out_shape=(o, o, o),
        in_specs=[blk(16), full((16, 128)), full((1, 128)),
                  full((128, 128)), full((128, 128)), full((1, 128))],
        out_specs=(blk(128), blk(128), blk(128)),
    )(nl4.reshape(NB, NRB, 16), WpK, bpK, WsK, WdK, bcK)


def _prep_edge_body(el16, W1K, b1K, W2K, b2K, ef4):
    t = jnp.maximum(_dot(el16[...], W1K[...]) + b1K[...], 0.0)
    e = _dot(t, W2K[...]) + b2K[...]
    ef4[...] = e.reshape(ef4.shape)


def _prep_edge(el16, W1K, b1K, W2K, b2K):
    RE = 2000
    g = (E // 16) // RE
    full = lambda s: pl.BlockSpec(s, lambda i: (0,) * len(s))
    return pl.pallas_call(
        _prep_edge_body,
        grid=(g,),
        out_shape=jax.ShapeDtypeStruct((E // 4, 128), jnp.float32),
        in_specs=[pl.BlockSpec((RE, 64), lambda i: (i, 0)),
                  full((64, 512)), full((1, 512)), full((512, 512)),
                  full((1, 512))],
        out_specs=pl.BlockSpec((4 * RE, 128), lambda i: (i, 0)),
    )(el16, W1K, b1K, W2K, b2K)


def _node_update_body(nf4, agg0, agg1, Wc1K, bc1K, Wc2K, bc2K, WsK, WdK, bcK,
                      nf4o, a4, d4):
    h = nf4[0] + agg0[0] + agg1[0]
    t = jnp.maximum(_dot(h, Wc1K[...]) + bc1K[...], 0.0)
    nf2 = jnp.maximum(_dot(t, Wc2K[...]) + bc2K[...], 0.0)
    nf4o[0] = nf2
    a4[0] = _dot(nf2, WsK[...])
    d4[0] = _dot(nf2, WdK[...]) + bcK[...]


def _node_update(nf4, agg4, Wc1K, bc1K, Wc2K, bc2K, WsK, WdK, bcK):
    full = lambda s: pl.BlockSpec(s, lambda i: (0,) * len(s))
    blk = lambda: pl.BlockSpec((1, NRB, 128), lambda i: (i, 0, 0))
    o = jax.ShapeDtypeStruct((NB, NRB, 128), jnp.float32)
    r3 = lambda x: x.reshape(NB, NRB, 128)
    return pl.pallas_call(
        _node_update_body,
        grid=(NB,),
        out_shape=(o, o, o),
        in_specs=[blk(), blk(), blk(),
                  full((128, 128)), full((1, 128)), full((128, 128)),
                  full((1, 128)), full((128, 128)), full((128, 128)),
                  full((1, 128))],
        out_specs=(blk(), blk(), blk()),
    )(r3(nf4), r3(agg4[0]), r3(agg4[1]), Wc1K, bc1K, Wc2K, bc2K, WsK, WdK,
      bcK)


def _node_final_body(nf4, agg0, agg1, Wc1K, bc1K, Wc2K, bc2K, WnhK, bnhK,
                     out):
    h = nf4[0] + agg0[0] + agg1[0]
    t = jnp.maximum(_dot(h, Wc1K[...]) + bc1K[...], 0.0)
    nf2 = jnp.maximum(_dot(t, Wc2K[...]) + bc2K[...], 0.0)
    out[0] = _dot(nf2, WnhK[...]) + bnhK[...]


def _node_final(nf4, agg4, Wc1K, bc1K, Wc2K, bc2K, WnhK, bnhK):
    full = lambda s: pl.BlockSpec(s, lambda i: (0,) * len(s))
    blk = lambda m: pl.BlockSpec((1, NRB, m), lambda i: (i, 0, 0))
    r3 = lambda x: x.reshape(NB, NRB, 128)
    return pl.pallas_call(
        _node_final_body,
        grid=(NB,),
        out_shape=jax.ShapeDtypeStruct((NB, NRB, 16), jnp.float32),
        in_specs=[blk(128), blk(128), blk(128),
                  full((128, 128)), full((1, 128)), full((128, 128)),
                  full((1, 128)), full((128, 16)), full((1, 16))],
        out_specs=blk(16),
    )(r3(nf4), r3(agg4[0]), r3(agg4[1]), Wc1K, bc1K, Wc2K, bc2K, WnhK, bnhK)


def _edge_head_body(ef4, WehK, behK, out):
    out[...] = _dot(ef4[...], WehK[...]) + behK[...]


def _edge_head(ef4, WehK, behK):
    RE = 8000
    g = (E // 4) // RE
    full = lambda s: pl.BlockSpec(s, lambda i: (0,) * len(s))
    return pl.pallas_call(
        _edge_head_body,
        grid=(g,),
        out_shape=jax.ShapeDtypeStruct((E // 4, 16), jnp.float32),
        in_specs=[pl.BlockSpec((RE, 128), lambda i: (i, 0)),
                  full((128, 16)), full((1, 16))],
        out_specs=pl.BlockSpec((RE, 16), lambda i: (i, 0)),
    )(ef4, WehK, behK)


def _sc_edge_body(A_hbm, C_hbm, D_hbm, ef_hbm, src_hbm, dst_hbm,
                  efo_hbm, agg_hbm,
                  sidx, didx, efb, ab, cb, db, aggsh, sem):
    c = lax.axis_index("c")
    s = lax.axis_index("s")
    w = s * SC_CORES + c

    zero = jnp.zeros((LANES,), jnp.float32)

    def _zrow(e, _):
        db[e, pl.ds(0, LANES)] = zero
        db[e, pl.ds(LANES, LANES)] = zero
        return _

    lax.fori_loop(0, K_EDGES, _zrow, None)
    base = s * ROWS_PER_TILE

    def _zero_rows(nrows):
        off = 0
        while off < nrows:
            sz = min(K_EDGES, nrows - off)
            pltpu.sync_copy(db.at[pl.ds(0, sz)],
                            aggsh.at[pl.ds(base + off, sz)])
            off += sz

    @pl.when(s < SC_SUBCORES - 1)
    def _():
        _zero_rows(ROWS_PER_TILE)

    @pl.when(s == SC_SUBCORES - 1)
    def _():
        _zero_rows(ROWS_LAST)

    plsc.subcore_barrier()

    def _chunk(t, _):
        chunk = w + t * SC_WORKERS

        @pl.when(chunk < NCHUNKS)
        def _():
            ebase = chunk * K_EDGES
            d_si = pltpu.async_copy(src_hbm.at[chunk], sidx, sem)
            d_di = pltpu.async_copy(dst_hbm.at[chunk], didx, sem)
            d_ef = pltpu.async_copy(ef_hbm.at[pl.ds(ebase, K_EDGES)], efb, sem)
            d_si.wait()
            d_di.wait()
            gathers = []
            for j in range(NSUB):
                gathers.append(pltpu.async_copy(
                    A_hbm.at[sidx.at[j]], ab.at[pl.ds(j * SUB, SUB)], sem))
                gathers.append(pltpu.async_copy(
                    C_hbm.at[sidx.at[j]], cb.at[pl.ds(j * SUB, SUB)], sem))
                gathers.append(pltpu.async_copy(
                    D_hbm.at[didx.at[j]], db.at[pl.ds(j * SUB, SUB)], sem))
            d_ef.wait()
            for g in gathers:
                g.wait()

            def _edge(e, _):
                a0 = ab[e, pl.ds(0, LANES)]
                a1 = ab[e, pl.ds(LANES, LANES)]
                f0 = efb[e, pl.ds(0, LANES)]
                f1 = efb[e, pl.ds(LANES, LANES)]
                d0 = db[e, pl.ds(0, LANES)]
                d1 = db[e, pl.ds(LANES, LANES)]
                g0 = jnp.maximum(f0 + a0 + d0, 0.0)
                g1 = jnp.maximum(f1 + a1 + d1, 0.0)
                efb[e, pl.ds(0, LANES)] = g0
                efb[e, pl.ds(LANES, LANES)] = g1
                c0 = cb[e, pl.ds(0, LANES)]
                c1 = cb[e, pl.ds(LANES, LANES)]
                db[e, pl.ds(0, LANES)] = jnp.maximum(c0 + g0, 0.0)
                db[e, pl.ds(LANES, LANES)] = jnp.maximum(c1 + g1, 0.0)
                return _

            lax.fori_loop(0, K_EDGES, _edge, None)

            for j in range(NSUB):
                pltpu.sync_copy(db.at[pl.ds(j * SUB, SUB)],
                                aggsh.at[didx.at[j]], add=True)
            pltpu.sync_copy(efb, efo_hbm.at[pl.ds(ebase, K_EDGES)])
        return _

    lax.fori_loop(0, T_STEPS, _chunk, None)
    plsc.subcore_barrier()

    @pl.when(s < SC_SUBCORES - 1)
    def _():
        pltpu.sync_copy(aggsh.at[pl.ds(base, ROWS_PER_TILE)],
                        agg_hbm.at[c, pl.ds(base, ROWS_PER_TILE)])

    @pl.when(s == SC_SUBCORES - 1)
    def _():
        pltpu.sync_copy(aggsh.at[pl.ds(base, ROWS_LAST)],
                        agg_hbm.at[c, pl.ds(base, ROWS_LAST)])


def _sc_edge_pass(a4, nf4, d4, ef4, src2, dst2):
    A_tab = a4.reshape(N, H)
    C_tab = nf4.reshape(N, H)
    D_tab = d4.reshape(N, H)
    ef = ef4.reshape(E, H)
    mesh = plsc.VectorSubcoreMesh(core_axis_name="c", subcore_axis_name="s")
    ef_o, agg = pl.kernel(
        _sc_edge_body,
        out_type=(jax.ShapeDtypeStruct((E, H), jnp.float32),
                  jax.ShapeDtypeStruct((SC_CORES, N, H), jnp.float32)),
        mesh=mesh,
        compiler_params=pltpu.CompilerParams(use_tc_tiling_on_sc=False),
        scratch_types=[
            pltpu.VMEM((NSUB, SUB), jnp.int32),
            pltpu.VMEM((NSUB, SUB), jnp.int32),
            pltpu.VMEM((K_EDGES, H), jnp.float32),
            pltpu.VMEM((K_EDGES, H), jnp.float32),
            pltpu.VMEM((K_EDGES, H), jnp.float32),
            pltpu.VMEM((K_EDGES, H), jnp.float32),
            pltpu.VMEM_SHARED((N, H), jnp.float32),
            pltpu.SemaphoreType.DMA,
        ],
    )(A_tab, C_tab, D_tab, ef, src2, dst2)
    return ef_o.reshape(E // 4, 128), agg.reshape(SC_CORES, N // 4, 128)


def kernel(node_logits, edge_logits, edge_index, W_proj, b_proj, W_e1, b_e1,
           W_e2, b_e2, W_ap, b_ap, W_v, b_v, W_o, b_o, W_c1, b_c1, W_c2,
           b_c2, W_nh, b_nh, W_eh, b_eh):
    src2 = edge_index[0].reshape(NCHUNKS, NSUB, SUB)
    dst2 = edge_index[1].reshape(NCHUNKS, NSUB, SUB)
    nl4 = node_logits.reshape(N // 4, 16)
    el16 = edge_logits.reshape(E // 16, 64)

    Ws, Wd, bc = _fold_weights(W_ap, b_ap, W_v, b_v, W_o, b_o)

    WpK = _kron(W_proj, 4)
    bpK = _tileb(b_proj, 4)
    WsK = _kron(Ws, 4)
    WdK = _kron(Wd, 4)
    bcK = _tileb(bc, 4)
    W1K = _kron(W_e1, 16)
    b1K = _tileb(b_e1, 16)
    W2K = _kron(W_e2, 16)
    b2K = _tileb(b_e2, 16)
    Wc1K = _kron(W_c1, 4)
    bc1K = _tileb(b_c1, 4)
    Wc2K = _kron(W_c2, 4)
    bc2K = _tileb(b_c2, 4)
    WnhK = _kron(W_nh, 4)
    bnhK = _tileb(b_nh, 4)
    WehK = _kron(W_eh, 4)
    behK = _tileb(b_eh, 4)

    nf4, a4, d4 = _prep_node(nl4, WpK, bpK, WsK, WdK, bcK)
    ef4 = _prep_edge(el16, W1K, b1K, W2K, b2K)

    ef4, agg4 = _sc_edge_pass(a4, nf4, d4, ef4, src2, dst2)
    nf4, a4, d4 = _node_update(nf4, agg4, Wc1K, bc1K, Wc2K, bc2K, WsK, WdK,
                               bcK)

    ef4, agg4 = _sc_edge_pass(a4, nf4, d4, ef4, src2, dst2)
    node_out = _node_final(nf4, agg4, Wc1K, bc1K, Wc2K, bc2K, WnhK, bnhK)
    edge_out = _edge_head(ef4, WehK, behK)
    return (node_out.reshape(N, NC), edge_out.reshape(E, EC))


# speedup vs baseline: 4.9315x; 1.3399x over previous
import jax
import jax.numpy as jnp
from jax import lax
from jax.experimental import pallas as pl
from jax.experimental.pallas import tpu as pltpu
from jax.experimental.pallas import tpu_sc as plsc

N = 50000
E = 800000
H = 32
NC = 4
EC = 4

SC_CORES = 2
SC_SUBCORES = 16
SC_WORKERS = SC_CORES * SC_SUBCORES
LANES = 16

K_EDGES = 200
SUB = 100
NSUB = K_EDGES // SUB
NCHUNKS = E // K_EDGES
T_STEPS = -(-NCHUNKS // SC_WORKERS)
ROWS_PER_TILE = 3128
ROWS_LAST = N - (SC_SUBCORES - 1) * ROWS_PER_TILE

_P = jax.lax.Precision.HIGHEST


def _dot(x, w):
    return jnp.dot(x, w, precision=_P, preferred_element_type=jnp.float32)


def _kron(w, p):
    return jnp.kron(jnp.eye(p, dtype=jnp.float32), w)


def _tileb(b, p):
    return jnp.tile(b.reshape(-1), p).reshape(1, -1)


def _fold_body(W_ap, b_ap, W_v, b_v, W_o, b_o, Ws, Wd, bc):
    wvo = _dot(W_v[...], W_o[...])
    Ws[...] = _dot(W_ap[pl.ds(0, H), :], wvo)
    Wd[...] = _dot(W_ap[pl.ds(H, H), :], wvo)
    bc[...] = _dot(_dot(b_ap[...], W_v[...]) + b_v[...], W_o[...]) + b_o[...]


def _fold_weights(W_ap, b_ap, W_v, b_v, W_o, b_o):
    full = lambda s: pl.BlockSpec(s, lambda: (0,) * len(s))
    return pl.pallas_call(
        _fold_body,
        out_shape=(jax.ShapeDtypeStruct((H, H), jnp.float32),
                   jax.ShapeDtypeStruct((H, H), jnp.float32),
                   jax.ShapeDtypeStruct((1, H), jnp.float32)),
        in_specs=[full((2 * H, H)), full((1, H)), full((H, H)), full((1, H)),
                  full((H, H)), full((1, H))],
        out_specs=(full((H, H)), full((H, H)), full((1, H))),
    )(W_ap, b_ap.reshape(1, H), W_v, b_v.reshape(1, H), W_o, b_o.reshape(1, H))


NB = 4
NRB = N // 4 // NB


def _prep_node_body(nl4, WpK, bpK, WsK, WdK, bcK, nf4, a4, d4):
    nf = _dot(nl4[0], WpK[...]) + bpK[...]
    nf4[0] = nf
    a4[0] = _dot(nf, WsK[...])
    d4[0] = _dot(nf, WdK[...]) + bcK[...]


def _prep_node(nl4, WpK, bpK, WsK, WdK, bcK):
    full = lambda s: pl.BlockSpec(s, lambda i: (0,) * len(s))
    blk = lambda m: pl.BlockSpec((1, NRB, m), lambda i: (i, 0, 0))
    o = jax.ShapeDtypeStruct((NB, NRB, 128), jnp.float32)
    return pl.pallas_call(
        _prep_node_body,
        grid=(NB,),
        out_shape=(o, o, o),
        in_specs=[blk(16), full((16, 128)), full((1, 128)),
                  full((128, 128)), full((128, 128)), full((1, 128))],
        out_specs=(blk(128), blk(128), blk(128)),
    )(nl4.reshape(NB, NRB, 16), WpK, bpK, WsK, WdK, bcK)


def _prep_edge_body(el16, W1K, b1K, W2K, b2K, ef4):
    t = jnp.maximum(_dot(el16[...], W1K[...]) + b1K[...], 0.0)
    e = _dot(t, W2K[...]) + b2K[...]
    ef4[...] = e.reshape(ef4.shape)


def _prep_edge(el16, W1K, b1K, W2K, b2K):
    RE = 2000
    g = (E // 16) // RE
    full = lambda s: pl.BlockSpec(s, lambda i: (0,) * len(s))
    return pl.pallas_call(
        _prep_edge_body,
        grid=(g,),
        out_shape=jax.ShapeDtypeStruct((E // 4, 128), jnp.float32),
        in_specs=[pl.BlockSpec((RE, 64), lambda i: (i, 0)),
                  full((64, 512)), full((1, 512)), full((512, 512)),
                  full((1, 512))],
        out_specs=pl.BlockSpec((4 * RE, 128), lambda i: (i, 0)),
    )(el16, W1K, b1K, W2K, b2K)


def _node_update_body(nf4, agg0, agg1, Wc1K, bc1K, Wc2K, bc2K, WsK, WdK, bcK,
                      nf4o, a4, d4):
    h = nf4[0] + agg0[0] + agg1[0]
    t = jnp.maximum(_dot(h, Wc1K[...]) + bc1K[...], 0.0)
    nf2 = jnp.maximum(_dot(t, Wc2K[...]) + bc2K[...], 0.0)
    nf4o[0] = nf2
    a4[0] = _dot(nf2, WsK[...])
    d4[0] = _dot(nf2, WdK[...]) + bcK[...]


def _node_update(nf4, agg4, Wc1K, bc1K, Wc2K, bc2K, WsK, WdK, bcK):
    full = lambda s: pl.BlockSpec(s, lambda i: (0,) * len(s))
    blk = lambda: pl.BlockSpec((1, NRB, 128), lambda i: (i, 0, 0))
    o = jax.ShapeDtypeStruct((NB, NRB, 128), jnp.float32)
    r3 = lambda x: x.reshape(NB, NRB, 128)
    return pl.pallas_call(
        _node_update_body,
        grid=(NB,),
        out_shape=(o, o, o),
        in_specs=[blk(), blk(), blk(),
                  full((128, 128)), full((1, 128)), full((128, 128)),
                  full((1, 128)), full((128, 128)), full((128, 128)),
                  full((1, 128))],
        out_specs=(blk(), blk(), blk()),
    )(r3(nf4), r3(agg4[0]), r3(agg4[1]), Wc1K, bc1K, Wc2K, bc2K, WsK, WdK,
      bcK)


def _node_final_body(nf4, agg0, agg1, Wc1K, bc1K, Wc2K, bc2K, WnhK, bnhK,
                     out):
    h = nf4[0] + agg0[0] + agg1[0]
    t = jnp.maximum(_dot(h, Wc1K[...]) + bc1K[...], 0.0)
    nf2 = jnp.maximum(_dot(t, Wc2K[...]) + bc2K[...], 0.0)
    out[0] = _dot(nf2, WnhK[...]) + bnhK[...]


def _node_final(nf4, agg4, Wc1K, bc1K, Wc2K, bc2K, WnhK, bnhK):
    full = lambda s: pl.BlockSpec(s, lambda i: (0,) * len(s))
    blk = lambda m: pl.BlockSpec((1, NRB, m), lambda i: (i, 0, 0))
    r3 = lambda x: x.reshape(NB, NRB, 128)
    return pl.pallas_call(
        _node_final_body,
        grid=(NB,),
        out_shape=jax.ShapeDtypeStruct((NB, NRB, 16), jnp.float32),
        in_specs=[blk(128), blk(128), blk(128),
                  full((128, 128)), full((1, 128)), full((128, 128)),
                  full((1, 128)), full((128, 16)), full((1, 16))],
        out_specs=blk(16),
    )(r3(nf4), r3(agg4[0]), r3(agg4[1]), Wc1K, bc1K, Wc2K, bc2K, WnhK, bnhK)


def _edge_head_body(ef4, WehP, behP, out):
    x3 = ef4[...].reshape(ef4.shape[0] // 4, 4, 128)
    acc = behP[...]
    for m in range(4):
        acc = acc + _dot(x3[:, m, :], WehP[m])
    out[...] = acc


def _edge_head(ef4, WehP, behP):
    RE = 8000
    g = (E // 4) // RE
    full = lambda s: pl.BlockSpec(s, lambda i: (0,) * len(s))
    return pl.pallas_call(
        _edge_head_body,
        grid=(g,),
        out_shape=jax.ShapeDtypeStruct((E // 16, 64), jnp.float32),
        in_specs=[pl.BlockSpec((RE, 128), lambda i: (i, 0)),
                  full((4, 128, 64)), full((1, 64))],
        out_specs=pl.BlockSpec((RE // 4, 64), lambda i: (i, 0)),
    )(ef4, WehP, behP)


def _sc_edge_body(A_hbm, C_hbm, D_hbm, ef_hbm, src_hbm, dst_hbm,
                  efo_hbm, agg_hbm,
                  sidx, didx, efb, ab, cb, db, aggsh, sem):
    c = lax.axis_index("c")
    s = lax.axis_index("s")
    w = s * SC_CORES + c

    zero = jnp.zeros((LANES,), jnp.float32)

    def _zrow(e, _):
        db[e, pl.ds(0, LANES)] = zero
        db[e, pl.ds(LANES, LANES)] = zero
        return _

    lax.fori_loop(0, K_EDGES, _zrow, None)
    base = s * ROWS_PER_TILE

    def _zero_rows(nrows):
        off = 0
        while off < nrows:
            sz = min(K_EDGES, nrows - off)
            pltpu.sync_copy(db.at[pl.ds(0, sz)],
                            aggsh.at[pl.ds(base + off, sz)])
            off += sz

    @pl.when(s < SC_SUBCORES - 1)
    def _():
        _zero_rows(ROWS_PER_TILE)

    @pl.when(s == SC_SUBCORES - 1)
    def _():
        _zero_rows(ROWS_LAST)

    plsc.subcore_barrier()

    def _chunk(t, _):
        chunk = w + t * SC_WORKERS

        @pl.when(chunk < NCHUNKS)
        def _():
            ebase = chunk * K_EDGES
            d_si = pltpu.async_copy(src_hbm.at[chunk], sidx, sem)
            d_di = pltpu.async_copy(dst_hbm.at[chunk], didx, sem)
            d_ef = pltpu.async_copy(ef_hbm.at[pl.ds(ebase, K_EDGES)], efb, sem)
            d_si.wait()
            d_di.wait()
            gathers = []
            for j in range(NSUB):
                gathers.append(pltpu.async_copy(
                    A_hbm.at[sidx.at[j]], ab.at[pl.ds(j * SUB, SUB)], sem))
                gathers.append(pltpu.async_copy(
                    C_hbm.at[sidx.at[j]], cb.at[pl.ds(j * SUB, SUB)], sem))
                gathers.append(pltpu.async_copy(
                    D_hbm.at[didx.at[j]], db.at[pl.ds(j * SUB, SUB)], sem))
            d_ef.wait()
            for g in gathers:
                g.wait()

            def _edge(e, _):
                a0 = ab[e, pl.ds(0, LANES)]
                a1 = ab[e, pl.ds(LANES, LANES)]
                f0 = efb[e, pl.ds(0, LANES)]
                f1 = efb[e, pl.ds(LANES, LANES)]
                d0 = db[e, pl.ds(0, LANES)]
                d1 = db[e, pl.ds(LANES, LANES)]
                g0 = jnp.maximum(f0 + a0 + d0, 0.0)
                g1 = jnp.maximum(f1 + a1 + d1, 0.0)
                efb[e, pl.ds(0, LANES)] = g0
                efb[e, pl.ds(LANES, LANES)] = g1
                c0 = cb[e, pl.ds(0, LANES)]
                c1 = cb[e, pl.ds(LANES, LANES)]
                db[e, pl.ds(0, LANES)] = jnp.maximum(c0 + g0, 0.0)
                db[e, pl.ds(LANES, LANES)] = jnp.maximum(c1 + g1, 0.0)
                return _

            lax.fori_loop(0, K_EDGES, _edge, None)

            for j in range(NSUB):
                pltpu.sync_copy(db.at[pl.ds(j * SUB, SUB)],
                                aggsh.at[didx.at[j]], add=True)
            pltpu.sync_copy(efb, efo_hbm.at[pl.ds(ebase, K_EDGES)])
        return _

    lax.fori_loop(0, T_STEPS, _chunk, None)
    plsc.subcore_barrier()

    @pl.when(s < SC_SUBCORES - 1)
    def _():
        pltpu.sync_copy(aggsh.at[pl.ds(base, ROWS_PER_TILE)],
                        agg_hbm.at[c, pl.ds(base, ROWS_PER_TILE)])

    @pl.when(s == SC_SUBCORES - 1)
    def _():
        pltpu.sync_copy(aggsh.at[pl.ds(base, ROWS_LAST)],
                        agg_hbm.at[c, pl.ds(base, ROWS_LAST)])


def _sc_edge_pass(a4, nf4, d4, ef4, src2, dst2):
    A_tab = a4.reshape(N, H)
    C_tab = nf4.reshape(N, H)
    D_tab = d4.reshape(N, H)
    ef = ef4.reshape(E, H)
    mesh = plsc.VectorSubcoreMesh(core_axis_name="c", subcore_axis_name="s")
    ef_o, agg = pl.kernel(
        _sc_edge_body,
        out_type=(jax.ShapeDtypeStruct((E, H), jnp.float32),
                  jax.ShapeDtypeStruct((SC_CORES, N, H), jnp.float32)),
        mesh=mesh,
        compiler_params=pltpu.CompilerParams(use_tc_tiling_on_sc=False),
        scratch_types=[
            pltpu.VMEM((NSUB, SUB), jnp.int32),
            pltpu.VMEM((NSUB, SUB), jnp.int32),
            pltpu.VMEM((K_EDGES, H), jnp.float32),
            pltpu.VMEM((K_EDGES, H), jnp.float32),
            pltpu.VMEM((K_EDGES, H), jnp.float32),
            pltpu.VMEM((K_EDGES, H), jnp.float32),
            pltpu.VMEM_SHARED((N, H), jnp.float32),
            pltpu.SemaphoreType.DMA,
        ],
    )(A_tab, C_tab, D_tab, ef, src2, dst2)
    return ef_o.reshape(E // 4, 128), agg.reshape(SC_CORES, N // 4, 128)


def kernel(node_logits, edge_logits, edge_index, W_proj, b_proj, W_e1, b_e1,
           W_e2, b_e2, W_ap, b_ap, W_v, b_v, W_o, b_o, W_c1, b_c1, W_c2,
           b_c2, W_nh, b_nh, W_eh, b_eh):
    src2 = edge_index[0].reshape(NCHUNKS, NSUB, SUB)
    dst2 = edge_index[1].reshape(NCHUNKS, NSUB, SUB)
    nl4 = node_logits.reshape(N // 4, 16)
    elT = edge_logits.T
    el16 = jnp.concatenate(
        [elT[f].reshape(E // 16, 16) for f in range(EC)], axis=1)

    Ws, Wd, bc = _fold_weights(W_ap, b_ap, W_v, b_v, W_o, b_o)

    WpK = _kron(W_proj, 4)
    bpK = _tileb(b_proj, 4)
    WsK = _kron(Ws, 4)
    WdK = _kron(Wd, 4)
    bcK = _tileb(bc, 4)
    W1K = jnp.einsum('jk,fc->fjkc', jnp.eye(16, dtype=jnp.float32),
                     W_e1).reshape(64, 512)
    b1K = _tileb(b_e1, 16)
    W2K = _kron(W_e2, 16)
    b2K = _tileb(b_e2, 16)
    Wc1K = _kron(W_c1, 4)
    bc1K = _tileb(b_c1, 4)
    Wc2K = _kron(W_c2, 4)
    bc2K = _tileb(b_c2, 4)
    WnhK = _kron(W_nh, 4)
    bnhK = _tileb(b_nh, 4)
    eye16 = jnp.eye(16, dtype=jnp.float32)
    WehP = jnp.stack([
        jnp.einsum('kf,ag->akfg', W_eh, eye16[4 * m:4 * m + 4]).reshape(
            128, 64)
        for m in range(4)])
    behP = jnp.repeat(b_eh, 16).reshape(1, 64)

    nf4, a4, d4 = _prep_node(nl4, WpK, bpK, WsK, WdK, bcK)
    ef4 = _prep_edge(el16, W1K, b1K, W2K, b2K)

    ef4, agg4 = _sc_edge_pass(a4, nf4, d4, ef4, src2, dst2)
    nf4, a4, d4 = _node_update(nf4, agg4, Wc1K, bc1K, Wc2K, bc2K, WsK, WdK,
                               bcK)

    ef4, agg4 = _sc_edge_pass(a4, nf4, d4, ef4, src2, dst2)
    node_out = _node_final(nf4, agg4, Wc1K, bc1K, Wc2K, bc2K, WnhK, bnhK)
    out16 = _edge_head(ef4, WehP, behP)
    edge_out = jnp.stack(
        [out16[:, 16 * f:16 * (f + 1)].reshape(E) for f in range(EC)],
        axis=1)
    return (node_out.reshape(N, NC), edge_out)
